# baseline (speedup 1.0000x reference)
import functools

import jax
import jax.numpy as jnp
from jax import lax
from jax.experimental import pallas as pl
from jax.experimental.pallas import tpu as pltpu
from jax.experimental.pallas import tpu_sc as plsc

F = 128
G = 64
NN = 10000
TILE = 256
GRID = 40
NP = TILE * GRID
EE = 320000
NWORK = 32
CK = 128
NB = 2
SB = 20
NSUP = 4
CH = SB * NSUP
EP = NWORK * CH * CK
RPS = NP // 16


def _sc_agg_body(h_hbm, src_hbm, dst_hbm, z_hbm, out_hbm,
                 src_v, dst_v, rows0, rows1, acc_sh,
                 rsem0, rsem1, isrc0, isrc1, idst0, idst1):
    rows = (rows0, rows1)
    rsems = (rsem0, rsem1)
    isrcs = (isrc0, isrc1)
    idsts = (idst0, idst1)
    c = lax.axis_index("c")
    sid = lax.axis_index("s")
    wid = sid * 2 + c
    pltpu.sync_copy(z_hbm, acc_sh.at[pl.ds(sid * RPS, RPS)])
    pltpu.sync_copy(src_hbm.at[wid, 0], src_v.at[0])
    pltpu.sync_copy(dst_hbm.at[wid, 0], dst_v.at[0])
    pltpu.async_copy(h_hbm.at[src_v.at[0, 0]], rows[0], rsems[0])
    plsc.subcore_barrier()

    for s in range(NSUP):
        slot = s % 2
        nxt = 1 - slot
        if s + 1 < NSUP:
            pltpu.async_copy(src_hbm.at[wid, s + 1], src_v.at[nxt],
                             isrcs[nxt])
            pltpu.async_copy(dst_hbm.at[wid, s + 1], dst_v.at[nxt],
                             idsts[nxt])

        def body(blk, carry, slot=slot):
            for b in range(NB):
                u = blk * NB + b
                pltpu.make_async_copy(h_hbm.at[src_v.at[slot, u]], rows[b],
                                      rsems[b]).wait()
                pltpu.async_copy(h_hbm.at[src_v.at[slot, u + 1]],
                                 rows[1 - b], rsems[1 - b])
                pltpu.sync_copy(rows[b], acc_sh.at[dst_v.at[slot, u]],
                                add=True)
            return carry

        lax.fori_loop(0, (SB - NB) // NB, body, 0)

        pltpu.make_async_copy(h_hbm.at[src_v.at[slot, SB - 2]], rows[0],
                              rsems[0]).wait()
        pltpu.async_copy(h_hbm.at[src_v.at[slot, SB - 1]], rows[1],
                         rsems[1])
        pltpu.sync_copy(rows[0], acc_sh.at[dst_v.at[slot, SB - 2]],
                        add=True)
        pltpu.make_async_copy(h_hbm.at[src_v.at[slot, SB - 1]], rows[1],
                              rsems[1]).wait()
        if s + 1 < NSUP:
            pltpu.make_async_copy(src_hbm.at[wid, 0], src_v.at[nxt],
                                  isrcs[nxt]).wait()
            pltpu.make_async_copy(dst_hbm.at[wid, 0], dst_v.at[nxt],
                                  idsts[nxt]).wait()
            pltpu.async_copy(h_hbm.at[src_v.at[nxt, 0]], rows[0], rsems[0])
        pltpu.sync_copy(rows[1], acc_sh.at[dst_v.at[slot, SB - 1]],
                        add=True)

    plsc.subcore_barrier()
    pltpu.sync_copy(acc_sh.at[pl.ds(sid * RPS, RPS)],
                    out_hbm.at[c, pl.ds(sid * RPS, RPS)])


def _make_agg(interpret=False):
    return pl.kernel(
        _sc_agg_body,
        out_type=jax.ShapeDtypeStruct((2, NP, F), jnp.float32),
        mesh=plsc.VectorSubcoreMesh(core_axis_name="c", subcore_axis_name="s",
                                    num_cores=2, num_subcores=16),
        scratch_types=[
            pltpu.VMEM((2, SB, CK), jnp.int32),
            pltpu.VMEM((2, SB, CK), jnp.int32),
            pltpu.VMEM((CK, F), jnp.float32),
            pltpu.VMEM((CK, F), jnp.float32),
            pltpu.VMEM_SHARED((NP, F), jnp.float32),
            pltpu.SemaphoreType.DMA,
            pltpu.SemaphoreType.DMA,
            pltpu.SemaphoreType.DMA,
            pltpu.SemaphoreType.DMA,
            pltpu.SemaphoreType.DMA,
            pltpu.SemaphoreType.DMA,
        ],
        interpret=interpret,
    )


def _stats(i, seg_ref, h, emb_ref, std_ref, a_ref, b_ref, gam_ref, bet_ref,
           gsum, gsq, ssum, ssq, cnt):
    @pl.when(i == 0)
    def _():
        gsum[...] = jnp.zeros_like(gsum)
        gsq[...] = jnp.zeros_like(gsq)
        ssum[...] = jnp.zeros_like(ssum)
        ssq[...] = jnp.zeros_like(ssq)
        cnt[...] = jnp.zeros_like(cnt)

    seg = seg_ref[...]
    gid = lax.broadcasted_iota(jnp.int32, (TILE, G), 1)
    mask = (seg == gid).astype(jnp.float32)
    rows = lax.broadcasted_iota(jnp.int32, (TILE, 1), 0) + i * TILE
    vm = (rows < NN).astype(jnp.float32)
    hm = h * vm
    gsum[...] += jnp.sum(hm, axis=0, keepdims=True)
    gsq[...] += jnp.sum(hm * h, axis=0, keepdims=True)
    dn = (((0,), (0,)), ((), ()))
    ssum[...] += lax.dot_general(mask, h, dn, preferred_element_type=jnp.float32)
    ssq[...] += lax.dot_general(mask, h * h, dn, preferred_element_type=jnp.float32)
    cnt[...] += lax.dot_general(mask, jnp.ones((TILE, 1), jnp.float32), dn,
                                preferred_element_type=jnp.float32)

    @pl.when(i == GRID - 1)
    def _():
        mu = gsum[...] * (1.0 / NN)
        var = gsq[...] * (1.0 / NN) - mu * mu
        a = gam_ref[...] * lax.rsqrt(var + 1e-5)
        b = bet_ref[...] - mu * a
        a_ref[...] = a
        b_ref[...] = b
        c = cnt[...]
        c1 = jnp.maximum(c, 1.0)
        ep = ssum[...] / c1
        emb_ref[...] = jnp.where(c > 0.0, ep * a + b, 0.0)
        sv = jnp.maximum(ssq[...] / c1 - ep * ep, 0.0)
        std_ref[...] = jnp.abs(a) * jnp.sqrt(sv)


def _first_body(x_ref, seg_ref, w_ref, bias_ref, gam_ref, bet_ref,
                h_ref, emb_ref, std_ref, a_ref, b_ref,
                gsum, gsq, ssum, ssq, cnt):
    i = pl.program_id(0)
    z = x_ref[...]
    h = lax.dot_general(z, w_ref[...], (((1,), (1,)), ((), ())),
                        preferred_element_type=jnp.float32) + bias_ref[...]
    h_ref[...] = h
    _stats(i, seg_ref, h, emb_ref, std_ref, a_ref, b_ref, gam_ref, bet_ref,
           gsum, gsq, ssum, ssq, cnt)


def _layer_body(h_in_ref, a0_ref, a1_ref, seg_ref, w1_ref, w2_ref,
                gam_ref, bet_ref,
                h_ref, emb_ref, std_ref, a_ref, b_ref,
                gsum, gsq, ssum, ssq, cnt):
    i = pl.program_id(0)
    z = h_in_ref[...] + a0_ref[...] + a1_ref[...]
    dn = (((1,), (1,)), ((), ()))
    t = jnp.maximum(lax.dot_general(z, w1_ref[...], dn,
                                    preferred_element_type=jnp.float32), 0.0)
    h = jnp.maximum(lax.dot_general(t, w2_ref[...], dn,
                                    preferred_element_type=jnp.float32), 0.0)
    h_ref[...] = h
    _stats(i, seg_ref, h, emb_ref, std_ref, a_ref, b_ref, gam_ref, bet_ref,
           gsum, gsq, ssum, ssq, cnt)


_ROW = lambda i: (i, 0)
_FIX = lambda i: (0, 0)
_STAGE_OUT_SHAPE = [
    jax.ShapeDtypeStruct((NP, F), jnp.float32),
    jax.ShapeDtypeStruct((G, F), jnp.float32),
    jax.ShapeDtypeStruct((G, F), jnp.float32),
    jax.ShapeDtypeStruct((1, F), jnp.float32),
    jax.ShapeDtypeStruct((1, F), jnp.float32),
]
_STAGE_OUT_SPECS = [
    pl.BlockSpec((TILE, F), _ROW),
    pl.BlockSpec((G, F), _FIX),
    pl.BlockSpec((G, F), _FIX),
    pl.BlockSpec((1, F), _FIX),
    pl.BlockSpec((1, F), _FIX),
]
_STAGE_SCRATCH = [
    pltpu.VMEM((1, F), jnp.float32),
    pltpu.VMEM((1, F), jnp.float32),
    pltpu.VMEM((G, F), jnp.float32),
    pltpu.VMEM((G, F), jnp.float32),
    pltpu.VMEM((G, 1), jnp.float32),
]


def _make_first(interpret=False):
    return pl.pallas_call(
        _first_body,
        grid=(GRID,),
        in_specs=[
            pl.BlockSpec((TILE, F), _ROW),
            pl.BlockSpec((TILE, 1), _ROW),
            pl.BlockSpec((F, F), _FIX),
            pl.BlockSpec((1, F), _FIX),
            pl.BlockSpec((1, F), _FIX),
            pl.BlockSpec((1, F), _FIX),
        ],
        out_specs=_STAGE_OUT_SPECS,
        out_shape=_STAGE_OUT_SHAPE,
        scratch_shapes=_STAGE_SCRATCH,
        interpret=interpret,
    )


def _make_layer(interpret=False):
    return pl.pallas_call(
        _layer_body,
        grid=(GRID,),
        in_specs=[
            pl.BlockSpec((TILE, F), _ROW),
            pl.BlockSpec((TILE, F), _ROW),
            pl.BlockSpec((TILE, F), _ROW),
            pl.BlockSpec((TILE, 1), _ROW),
            pl.BlockSpec((F, F), _FIX),
            pl.BlockSpec((F, F), _FIX),
            pl.BlockSpec((1, F), _FIX),
            pl.BlockSpec((1, F), _FIX),
        ],
        out_specs=_STAGE_OUT_SPECS,
        out_shape=_STAGE_OUT_SHAPE,
        scratch_shapes=_STAGE_SCRATCH,
        interpret=interpret,
    )


def _norm_body(h_ref, a_ref, b_ref, o_ref):
    o_ref[...] = h_ref[...] * a_ref[...] + b_ref[...]


def _make_norm(interpret=False):
    return pl.pallas_call(
        _norm_body,
        grid=(GRID,),
        in_specs=[
            pl.BlockSpec((TILE, F), _ROW),
            pl.BlockSpec((1, F), _FIX),
            pl.BlockSpec((1, F), _FIX),
        ],
        out_specs=pl.BlockSpec((TILE, F), _ROW),
        out_shape=jax.ShapeDtypeStruct((NP, F), jnp.float32),
        interpret=interpret,
    )


_get_agg = functools.lru_cache(maxsize=None)(_make_agg)
_first_call = _make_first()
_layer_call = _make_layer()
_norm_call = _make_norm()


def kernel(x, edge_index, batch, Wt, bt, g0, beta0, W1s, W2s, gs, bs):
    f32 = jnp.float32
    xp = jnp.zeros((NP, F), f32).at[:NN].set(x)
    segp = jnp.full((NP, 1), G, jnp.int32).at[:NN, 0].set(batch)
    pad = EP - EE
    srcp = jnp.concatenate(
        [edge_index[0], jnp.zeros((pad,), jnp.int32)]
    ).reshape(NWORK, NSUP, SB, CK)
    dstp = jnp.concatenate(
        [edge_index[1], jnp.full((pad,), NN, jnp.int32)]
    ).reshape(NWORK, NSUP, SB, CK)
    zrows = jnp.zeros((RPS, F), f32)

    h_pre, emb, std, a, b = _first_call(
        xp, segp, Wt, bt.reshape(1, F), g0.reshape(1, F), beta0.reshape(1, F))
    embeds, stds = [emb], [std]
    for i in range(3):
        h_new = _norm_call(h_pre, a, b)
        aggs = _get_agg()(h_new, srcp, dstp, zrows)
        h_pre, emb, std, a, b = _layer_call(
            h_new, aggs[0], aggs[1], segp, W1s[i], W2s[i],
            gs[i].reshape(1, F), bs[i].reshape(1, F))
        embeds.append(emb)
        stds.append(std)
    return jnp.stack(embeds), jnp.stack(stds)


# speedup vs baseline: 1.0647x; 1.0647x over previous
import functools

import jax
import jax.numpy as jnp
from jax import lax
from jax.experimental import pallas as pl
from jax.experimental.pallas import tpu as pltpu
from jax.experimental.pallas import tpu_sc as plsc

F = 128
G = 64
NN = 10000
TILE = 256
GRID = 40
NP = TILE * GRID
EE = 320000
NWORK = 32
CK = 128
CH = 80
EP = NWORK * CH * CK
RPS = NP // 16


def _sc_agg_body(h_hbm, src_hbm, dst_hbm, z_hbm, out_hbm,
                 src_v, dst_v, rows_v, acc_sh, sem):
    c = lax.axis_index("c")
    sid = lax.axis_index("s")
    wid = sid * 2 + c
    pltpu.sync_copy(z_hbm, acc_sh.at[pl.ds(sid * RPS, RPS)])
    pltpu.sync_copy(src_hbm.at[wid], src_v)
    pltpu.sync_copy(dst_hbm.at[wid], dst_v)
    plsc.subcore_barrier()

    def body(g, carry):
        pltpu.async_copy(h_hbm.at[src_v.at[g]], rows_v, sem).wait()
        pltpu.sync_copy(rows_v, acc_sh.at[dst_v.at[g]], add=True)
        return carry

    lax.fori_loop(0, CH, body, 0)
    plsc.subcore_barrier()
    pltpu.sync_copy(acc_sh.at[pl.ds(sid * RPS, RPS)],
                    out_hbm.at[c, pl.ds(sid * RPS, RPS)])


def _make_agg(interpret=False):
    return pl.kernel(
        _sc_agg_body,
        out_type=jax.ShapeDtypeStruct((2, NP, F), jnp.float32),
        mesh=plsc.VectorSubcoreMesh(core_axis_name="c", subcore_axis_name="s",
                                    num_cores=2, num_subcores=16),
        scratch_types=[
            pltpu.VMEM((CH, CK), jnp.int32),
            pltpu.VMEM((CH, CK), jnp.int32),
            pltpu.VMEM((CK, F), jnp.float32),
            pltpu.VMEM_SHARED((NP, F), jnp.float32),
            pltpu.SemaphoreType.DMA,
        ],
        interpret=interpret,
    )


def _stats(i, seg_ref, h, emb_ref, std_ref, a_ref, b_ref, gam_ref, bet_ref,
           gsum, gsq, ssum, ssq, cnt):
    @pl.when(i == 0)
    def _():
        gsum[...] = jnp.zeros_like(gsum)
        gsq[...] = jnp.zeros_like(gsq)
        ssum[...] = jnp.zeros_like(ssum)
        ssq[...] = jnp.zeros_like(ssq)
        cnt[...] = jnp.zeros_like(cnt)

    seg = seg_ref[...]
    gid = lax.broadcasted_iota(jnp.int32, (TILE, G), 1)
    mask = (seg == gid).astype(jnp.float32)
    rows = lax.broadcasted_iota(jnp.int32, (TILE, 1), 0) + i * TILE
    vm = (rows < NN).astype(jnp.float32)
    hm = h * vm
    gsum[...] += jnp.sum(hm, axis=0, keepdims=True)
    gsq[...] += jnp.sum(hm * h, axis=0, keepdims=True)
    dn = (((0,), (0,)), ((), ()))
    ssum[...] += lax.dot_general(mask, h, dn, preferred_element_type=jnp.float32)
    ssq[...] += lax.dot_general(mask, h * h, dn, preferred_element_type=jnp.float32)
    cnt[...] += lax.dot_general(mask, jnp.ones((TILE, 1), jnp.float32), dn,
                                preferred_element_type=jnp.float32)

    @pl.when(i == GRID - 1)
    def _():
        mu = gsum[...] * (1.0 / NN)
        var = gsq[...] * (1.0 / NN) - mu * mu
        a = gam_ref[...] * lax.rsqrt(var + 1e-5)
        b = bet_ref[...] - mu * a
        a_ref[...] = a
        b_ref[...] = b
        c = cnt[...]
        c1 = jnp.maximum(c, 1.0)
        ep = ssum[...] / c1
        emb_ref[...] = jnp.where(c > 0.0, ep * a + b, 0.0)
        sv = jnp.maximum(ssq[...] / c1 - ep * ep, 0.0)
        std_ref[...] = jnp.abs(a) * jnp.sqrt(sv)


def _first_body(x_ref, seg_ref, w_ref, bias_ref, gam_ref, bet_ref,
                h_ref, emb_ref, std_ref, a_ref, b_ref,
                gsum, gsq, ssum, ssq, cnt):
    i = pl.program_id(0)
    z = x_ref[...]
    h = lax.dot_general(z, w_ref[...], (((1,), (1,)), ((), ())),
                        preferred_element_type=jnp.float32) + bias_ref[...]
    h_ref[...] = h
    _stats(i, seg_ref, h, emb_ref, std_ref, a_ref, b_ref, gam_ref, bet_ref,
           gsum, gsq, ssum, ssq, cnt)


def _layer_body(h_in_ref, a0_ref, a1_ref, seg_ref, w1_ref, w2_ref,
                gam_ref, bet_ref,
                h_ref, emb_ref, std_ref, a_ref, b_ref,
                gsum, gsq, ssum, ssq, cnt):
    i = pl.program_id(0)
    z = h_in_ref[...] + a0_ref[...] + a1_ref[...]
    dn = (((1,), (1,)), ((), ()))
    t = jnp.maximum(lax.dot_general(z, w1_ref[...], dn,
                                    preferred_element_type=jnp.float32), 0.0)
    h = jnp.maximum(lax.dot_general(t, w2_ref[...], dn,
                                    preferred_element_type=jnp.float32), 0.0)
    h_ref[...] = h
    _stats(i, seg_ref, h, emb_ref, std_ref, a_ref, b_ref, gam_ref, bet_ref,
           gsum, gsq, ssum, ssq, cnt)


_ROW = lambda i: (i, 0)
_FIX = lambda i: (0, 0)
_STAGE_OUT_SHAPE = [
    jax.ShapeDtypeStruct((NP, F), jnp.float32),
    jax.ShapeDtypeStruct((G, F), jnp.float32),
    jax.ShapeDtypeStruct((G, F), jnp.float32),
    jax.ShapeDtypeStruct((1, F), jnp.float32),
    jax.ShapeDtypeStruct((1, F), jnp.float32),
]
_STAGE_OUT_SPECS = [
    pl.BlockSpec((TILE, F), _ROW),
    pl.BlockSpec((G, F), _FIX),
    pl.BlockSpec((G, F), _FIX),
    pl.BlockSpec((1, F), _FIX),
    pl.BlockSpec((1, F), _FIX),
]
_STAGE_SCRATCH = [
    pltpu.VMEM((1, F), jnp.float32),
    pltpu.VMEM((1, F), jnp.float32),
    pltpu.VMEM((G, F), jnp.float32),
    pltpu.VMEM((G, F), jnp.float32),
    pltpu.VMEM((G, 1), jnp.float32),
]


def _make_first(interpret=False):
    return pl.pallas_call(
        _first_body,
        grid=(GRID,),
        in_specs=[
            pl.BlockSpec((TILE, F), _ROW),
            pl.BlockSpec((TILE, 1), _ROW),
            pl.BlockSpec((F, F), _FIX),
            pl.BlockSpec((1, F), _FIX),
            pl.BlockSpec((1, F), _FIX),
            pl.BlockSpec((1, F), _FIX),
        ],
        out_specs=_STAGE_OUT_SPECS,
        out_shape=_STAGE_OUT_SHAPE,
        scratch_shapes=_STAGE_SCRATCH,
        interpret=interpret,
    )


def _make_layer(interpret=False):
    return pl.pallas_call(
        _layer_body,
        grid=(GRID,),
        in_specs=[
            pl.BlockSpec((TILE, F), _ROW),
            pl.BlockSpec((TILE, F), _ROW),
            pl.BlockSpec((TILE, F), _ROW),
            pl.BlockSpec((TILE, 1), _ROW),
            pl.BlockSpec((F, F), _FIX),
            pl.BlockSpec((F, F), _FIX),
            pl.BlockSpec((1, F), _FIX),
            pl.BlockSpec((1, F), _FIX),
        ],
        out_specs=_STAGE_OUT_SPECS,
        out_shape=_STAGE_OUT_SHAPE,
        scratch_shapes=_STAGE_SCRATCH,
        interpret=interpret,
    )


def _norm_body(h_ref, a_ref, b_ref, o_ref):
    o_ref[...] = h_ref[...] * a_ref[...] + b_ref[...]


def _make_norm(interpret=False):
    return pl.pallas_call(
        _norm_body,
        grid=(GRID,),
        in_specs=[
            pl.BlockSpec((TILE, F), _ROW),
            pl.BlockSpec((1, F), _FIX),
            pl.BlockSpec((1, F), _FIX),
        ],
        out_specs=pl.BlockSpec((TILE, F), _ROW),
        out_shape=jax.ShapeDtypeStruct((NP, F), jnp.float32),
        interpret=interpret,
    )


_get_agg = functools.lru_cache(maxsize=None)(_make_agg)
_first_call = _make_first()
_layer_call = _make_layer()
_norm_call = _make_norm()


def kernel(x, edge_index, batch, Wt, bt, g0, beta0, W1s, W2s, gs, bs):
    f32 = jnp.float32
    xp = jnp.zeros((NP, F), f32).at[:NN].set(x)
    segp = jnp.full((NP, 1), G, jnp.int32).at[:NN, 0].set(batch)
    pad = EP - EE
    srcp = jnp.concatenate(
        [edge_index[0], jnp.zeros((pad,), jnp.int32)]
    ).reshape(NWORK, CH, CK)
    dstp = jnp.concatenate(
        [edge_index[1], jnp.full((pad,), NN, jnp.int32)]
    ).reshape(NWORK, CH, CK)
    zrows = jnp.zeros((RPS, F), f32)

    h_pre, emb, std, a, b = _first_call(
        xp, segp, Wt, bt.reshape(1, F), g0.reshape(1, F), beta0.reshape(1, F))
    embeds, stds = [emb], [std]
    for i in range(3):
        h_new = _norm_call(h_pre, a, b)
        aggs = _get_agg()(h_new, srcp, dstp, zrows)
        h_pre, emb, std, a, b = _layer_call(
            h_new, aggs[0], aggs[1], segp, W1s[i], W2s[i],
            gs[i].reshape(1, F), bs[i].reshape(1, F))
        embeds.append(emb)
        stds.append(std)
    return jnp.stack(embeds), jnp.stack(stds)


# speedup vs baseline: 2.5273x; 2.3737x over previous
import functools

import jax
import jax.numpy as jnp
from jax import lax
from jax.experimental import pallas as pl
from jax.experimental.pallas import tpu as pltpu
from jax.experimental.pallas import tpu_sc as plsc

F = 128
G = 64
NN = 10000
TILE = 256
GRID = 40
NP = TILE * GRID
EE = 320000
NWORK = 32
CK = 128
CH = 80
EP = NWORK * CH * CK
RPS = NP // 16


def _sc_agg_body(h_hbm, src_hbm, dst_hbm, z_hbm, out_hbm,
                 src_v, dst_v, rows_v, acc_sh, sem):
    c = lax.axis_index("c")
    sid = lax.axis_index("s")
    wid = sid * 2 + c
    pltpu.sync_copy(z_hbm, acc_sh.at[pl.ds(sid * RPS, RPS)])
    pltpu.sync_copy(src_hbm.at[wid], src_v)
    pltpu.sync_copy(dst_hbm.at[wid], dst_v)
    plsc.subcore_barrier()

    def body(g, carry):
        pltpu.async_copy(h_hbm.at[src_v.at[g]], rows_v, sem).wait()
        pltpu.sync_copy(rows_v, acc_sh.at[dst_v.at[g]], add=True)
        return carry

    lax.fori_loop(0, CH, body, 0)
    plsc.subcore_barrier()
    pltpu.sync_copy(acc_sh.at[pl.ds(sid * RPS, RPS)],
                    out_hbm.at[c, pl.ds(sid * RPS, RPS)])


def _make_agg(interpret=False):
    return pl.kernel(
        _sc_agg_body,
        out_type=jax.ShapeDtypeStruct((2, NP, F), jnp.float32),
        mesh=plsc.VectorSubcoreMesh(core_axis_name="c", subcore_axis_name="s",
                                    num_cores=2, num_subcores=16),
        scratch_types=[
            pltpu.VMEM((CH, CK), jnp.int32),
            pltpu.VMEM((CH, CK), jnp.int32),
            pltpu.VMEM((CK, F), jnp.float32),
            pltpu.VMEM_SHARED((NP, F), jnp.float32),
            pltpu.SemaphoreType.DMA,
        ],
        interpret=interpret,
    )


def _stats(i, seg_ref, h, emb_ref, std_ref, a_ref, b_ref, gam_ref, bet_ref,
           gsum, gsq, ssum, ssq, cnt):
    @pl.when(i == 0)
    def _():
        gsum[...] = jnp.zeros_like(gsum)
        gsq[...] = jnp.zeros_like(gsq)
        ssum[...] = jnp.zeros_like(ssum)
        ssq[...] = jnp.zeros_like(ssq)
        cnt[...] = jnp.zeros_like(cnt)

    seg = seg_ref[...]
    gid = lax.broadcasted_iota(jnp.int32, (TILE, G), 1)
    mask = (seg == gid).astype(jnp.float32)
    rows = lax.broadcasted_iota(jnp.int32, (TILE, 1), 0) + i * TILE
    vm = (rows < NN).astype(jnp.float32)
    hm = h * vm
    gsum[...] += jnp.sum(hm, axis=0, keepdims=True)
    gsq[...] += jnp.sum(hm * h, axis=0, keepdims=True)
    dn = (((0,), (0,)), ((), ()))
    ssum[...] += lax.dot_general(mask, h, dn, preferred_element_type=jnp.float32)
    ssq[...] += lax.dot_general(mask, h * h, dn, preferred_element_type=jnp.float32)
    cnt[...] += lax.dot_general(mask, jnp.ones((TILE, 1), jnp.float32), dn,
                                preferred_element_type=jnp.float32)

    @pl.when(i == GRID - 1)
    def _():
        mu = gsum[...] * (1.0 / NN)
        var = gsq[...] * (1.0 / NN) - mu * mu
        a = gam_ref[...] * lax.rsqrt(var + 1e-5)
        b = bet_ref[...] - mu * a
        a_ref[...] = a
        b_ref[...] = b
        c = cnt[...]
        c1 = jnp.maximum(c, 1.0)
        ep = ssum[...] / c1
        emb_ref[...] = jnp.where(c > 0.0, ep * a + b, 0.0)
        sv = jnp.maximum(ssq[...] / c1 - ep * ep, 0.0)
        std_ref[...] = jnp.abs(a) * jnp.sqrt(sv)


def _first_body(x_ref, seg_ref, w_ref, bias_ref, gam_ref, bet_ref,
                h_ref, emb_ref, std_ref, a_ref, b_ref,
                gsum, gsq, ssum, ssq, cnt):
    i = pl.program_id(0)
    z = x_ref[...]
    h = lax.dot_general(z, w_ref[...], (((1,), (1,)), ((), ())),
                        preferred_element_type=jnp.float32) + bias_ref[...]
    h_ref[...] = h
    _stats(i, seg_ref, h, emb_ref, std_ref, a_ref, b_ref, gam_ref, bet_ref,
           gsum, gsq, ssum, ssq, cnt)


def _layer_body(h_in_ref, a0_ref, a1_ref, seg_ref, w1_ref, w2_ref,
                gam_ref, bet_ref,
                h_ref, emb_ref, std_ref, a_ref, b_ref,
                gsum, gsq, ssum, ssq, cnt):
    i = pl.program_id(0)
    z = h_in_ref[...] + a0_ref[...] + a1_ref[...]
    dn = (((1,), (1,)), ((), ()))
    t = jnp.maximum(lax.dot_general(z, w1_ref[...], dn,
                                    preferred_element_type=jnp.float32), 0.0)
    h = jnp.maximum(lax.dot_general(t, w2_ref[...], dn,
                                    preferred_element_type=jnp.float32), 0.0)
    h_ref[...] = h
    _stats(i, seg_ref, h, emb_ref, std_ref, a_ref, b_ref, gam_ref, bet_ref,
           gsum, gsq, ssum, ssq, cnt)


_ROW = lambda i: (i, 0)
_FIX = lambda i: (0, 0)
_STAGE_OUT_SHAPE = [
    jax.ShapeDtypeStruct((NP, F), jnp.float32),
    jax.ShapeDtypeStruct((G, F), jnp.float32),
    jax.ShapeDtypeStruct((G, F), jnp.float32),
    jax.ShapeDtypeStruct((1, F), jnp.float32),
    jax.ShapeDtypeStruct((1, F), jnp.float32),
]
_STAGE_OUT_SPECS = [
    pl.BlockSpec((TILE, F), _ROW),
    pl.BlockSpec((G, F), _FIX),
    pl.BlockSpec((G, F), _FIX),
    pl.BlockSpec((1, F), _FIX),
    pl.BlockSpec((1, F), _FIX),
]
_STAGE_SCRATCH = [
    pltpu.VMEM((1, F), jnp.float32),
    pltpu.VMEM((1, F), jnp.float32),
    pltpu.VMEM((G, F), jnp.float32),
    pltpu.VMEM((G, F), jnp.float32),
    pltpu.VMEM((G, 1), jnp.float32),
]


def _make_first(interpret=False):
    return pl.pallas_call(
        _first_body,
        grid=(GRID,),
        in_specs=[
            pl.BlockSpec((TILE, F), _ROW),
            pl.BlockSpec((TILE, 1), _ROW),
            pl.BlockSpec((F, F), _FIX),
            pl.BlockSpec((1, F), _FIX),
            pl.BlockSpec((1, F), _FIX),
            pl.BlockSpec((1, F), _FIX),
        ],
        out_specs=_STAGE_OUT_SPECS,
        out_shape=_STAGE_OUT_SHAPE,
        scratch_shapes=_STAGE_SCRATCH,
        interpret=interpret,
    )


def _make_layer(interpret=False):
    return pl.pallas_call(
        _layer_body,
        grid=(GRID,),
        in_specs=[
            pl.BlockSpec((TILE, F), _ROW),
            pl.BlockSpec((TILE, F), _ROW),
            pl.BlockSpec((TILE, F), _ROW),
            pl.BlockSpec((TILE, 1), _ROW),
            pl.BlockSpec((F, F), _FIX),
            pl.BlockSpec((F, F), _FIX),
            pl.BlockSpec((1, F), _FIX),
            pl.BlockSpec((1, F), _FIX),
        ],
        out_specs=_STAGE_OUT_SPECS,
        out_shape=_STAGE_OUT_SHAPE,
        scratch_shapes=_STAGE_SCRATCH,
        interpret=interpret,
    )


def _norm_body(h_ref, a_ref, b_ref, o_ref):
    o_ref[...] = h_ref[...] * a_ref[...] + b_ref[...]


def _make_norm(interpret=False):
    return pl.pallas_call(
        _norm_body,
        grid=(GRID,),
        in_specs=[
            pl.BlockSpec((TILE, F), _ROW),
            pl.BlockSpec((1, F), _FIX),
            pl.BlockSpec((1, F), _FIX),
        ],
        out_specs=pl.BlockSpec((TILE, F), _ROW),
        out_shape=jax.ShapeDtypeStruct((NP, F), jnp.float32),
        interpret=interpret,
    )


_get_agg = functools.lru_cache(maxsize=None)(_make_agg)
_first_call = _make_first()
_layer_call = _make_layer()
_norm_call = _make_norm()


def kernel(x, edge_index, batch, Wt, bt, g0, beta0, W1s, W2s, gs, bs):
    f32 = jnp.float32
    xp = jnp.zeros((NP, F), f32).at[:NN].set(x)
    segp = jnp.full((NP, 1), G, jnp.int32).at[:NN, 0].set(batch)
    pad = EP - EE
    pad_dst = NN + (jnp.arange(pad, dtype=jnp.int32) % (NP - NN))
    pad_src = jnp.arange(pad, dtype=jnp.int32) % NN
    srcp = jnp.concatenate(
        [edge_index[0], pad_src]).reshape(NWORK, CH, CK)
    dstp = jnp.concatenate(
        [edge_index[1], pad_dst]).reshape(NWORK, CH, CK)
    zrows = jnp.zeros((RPS, F), f32)

    h_pre, emb, std, a, b = _first_call(
        xp, segp, Wt, bt.reshape(1, F), g0.reshape(1, F), beta0.reshape(1, F))
    embeds, stds = [emb], [std]
    for i in range(3):
        h_new = _norm_call(h_pre, a, b)
        aggs = _get_agg()(h_new, srcp, dstp, zrows)
        h_pre, emb, std, a, b = _layer_call(
            h_new, aggs[0], aggs[1], segp, W1s[i], W2s[i],
            gs[i].reshape(1, F), bs[i].reshape(1, F))
        embeds.append(emb)
        stds.append(std)
    return jnp.stack(embeds), jnp.stack(stds)


# speedup vs baseline: 3.0046x; 1.1888x over previous
import functools

import jax
import jax.numpy as jnp
from jax import lax
from jax.experimental import pallas as pl
from jax.experimental.pallas import tpu as pltpu
from jax.experimental.pallas import tpu_sc as plsc

F = 128
G = 64
NN = 10000
TILE = 256
GRID = 40
NP = TILE * GRID
EE = 320000
NWORK = 32
CK = 128
NB = 2
SB = 20
NSUP = 4
CH = SB * NSUP
EP = NWORK * CH * CK
RPS = NP // 16


def _sc_agg_body(h_hbm, src_hbm, dst_hbm, z_hbm, out_hbm,
                 src_v, dst_v, rows0, rows1, acc_sh,
                 rsem0, rsem1, isrc0, isrc1, idst0, idst1):
    rows = (rows0, rows1)
    rsems = (rsem0, rsem1)
    isrcs = (isrc0, isrc1)
    idsts = (idst0, idst1)
    c = lax.axis_index("c")
    sid = lax.axis_index("s")
    wid = sid * 2 + c
    pltpu.sync_copy(z_hbm, acc_sh.at[pl.ds(sid * RPS, RPS)])
    pltpu.sync_copy(src_hbm.at[wid, 0], src_v.at[0])
    pltpu.sync_copy(dst_hbm.at[wid, 0], dst_v.at[0])
    pltpu.async_copy(h_hbm.at[src_v.at[0, 0]], rows[0], rsems[0])
    plsc.subcore_barrier()

    for s in range(NSUP):
        slot = s % 2
        nxt = 1 - slot
        if s + 1 < NSUP:
            pltpu.async_copy(src_hbm.at[wid, s + 1], src_v.at[nxt],
                             isrcs[nxt])
            pltpu.async_copy(dst_hbm.at[wid, s + 1], dst_v.at[nxt],
                             idsts[nxt])

        def body(blk, carry, slot=slot):
            for b in range(NB):
                u = blk * NB + b
                pltpu.make_async_copy(h_hbm.at[src_v.at[slot, u]], rows[b],
                                      rsems[b]).wait()
                pltpu.async_copy(h_hbm.at[src_v.at[slot, u + 1]],
                                 rows[1 - b], rsems[1 - b])
                pltpu.sync_copy(rows[b], acc_sh.at[dst_v.at[slot, u]],
                                add=True)
            return carry

        lax.fori_loop(0, (SB - NB) // NB, body, 0)

        pltpu.make_async_copy(h_hbm.at[src_v.at[slot, SB - 2]], rows[0],
                              rsems[0]).wait()
        pltpu.async_copy(h_hbm.at[src_v.at[slot, SB - 1]], rows[1],
                         rsems[1])
        pltpu.sync_copy(rows[0], acc_sh.at[dst_v.at[slot, SB - 2]],
                        add=True)
        pltpu.make_async_copy(h_hbm.at[src_v.at[slot, SB - 1]], rows[1],
                              rsems[1]).wait()
        if s + 1 < NSUP:
            pltpu.make_async_copy(src_hbm.at[wid, 0], src_v.at[nxt],
                                  isrcs[nxt]).wait()
            pltpu.make_async_copy(dst_hbm.at[wid, 0], dst_v.at[nxt],
                                  idsts[nxt]).wait()
            pltpu.async_copy(h_hbm.at[src_v.at[nxt, 0]], rows[0], rsems[0])
        pltpu.sync_copy(rows[1], acc_sh.at[dst_v.at[slot, SB - 1]],
                        add=True)

    plsc.subcore_barrier()
    pltpu.sync_copy(acc_sh.at[pl.ds(sid * RPS, RPS)],
                    out_hbm.at[c, pl.ds(sid * RPS, RPS)])


def _make_agg(interpret=False):
    return pl.kernel(
        _sc_agg_body,
        out_type=jax.ShapeDtypeStruct((2, NP, F), jnp.float32),
        mesh=plsc.VectorSubcoreMesh(core_axis_name="c", subcore_axis_name="s",
                                    num_cores=2, num_subcores=16),
        scratch_types=[
            pltpu.VMEM((2, SB, CK), jnp.int32),
            pltpu.VMEM((2, SB, CK), jnp.int32),
            pltpu.VMEM((CK, F), jnp.float32),
            pltpu.VMEM((CK, F), jnp.float32),
            pltpu.VMEM_SHARED((NP, F), jnp.float32),
            pltpu.SemaphoreType.DMA,
            pltpu.SemaphoreType.DMA,
            pltpu.SemaphoreType.DMA,
            pltpu.SemaphoreType.DMA,
            pltpu.SemaphoreType.DMA,
            pltpu.SemaphoreType.DMA,
        ],
        interpret=interpret,
    )


def _stats(i, seg_ref, h, emb_ref, std_ref, a_ref, b_ref, gam_ref, bet_ref,
           gsum, gsq, ssum, ssq, cnt):
    @pl.when(i == 0)
    def _():
        gsum[...] = jnp.zeros_like(gsum)
        gsq[...] = jnp.zeros_like(gsq)
        ssum[...] = jnp.zeros_like(ssum)
        ssq[...] = jnp.zeros_like(ssq)
        cnt[...] = jnp.zeros_like(cnt)

    seg = seg_ref[...]
    gid = lax.broadcasted_iota(jnp.int32, (TILE, G), 1)
    mask = (seg == gid).astype(jnp.float32)
    rows = lax.broadcasted_iota(jnp.int32, (TILE, 1), 0) + i * TILE
    vm = (rows < NN).astype(jnp.float32)
    hm = h * vm
    gsum[...] += jnp.sum(hm, axis=0, keepdims=True)
    gsq[...] += jnp.sum(hm * h, axis=0, keepdims=True)
    dn = (((0,), (0,)), ((), ()))
    ssum[...] += lax.dot_general(mask, h, dn, preferred_element_type=jnp.float32)
    ssq[...] += lax.dot_general(mask, h * h, dn, preferred_element_type=jnp.float32)
    cnt[...] += lax.dot_general(mask, jnp.ones((TILE, 1), jnp.float32), dn,
                                preferred_element_type=jnp.float32)

    @pl.when(i == GRID - 1)
    def _():
        mu = gsum[...] * (1.0 / NN)
        var = gsq[...] * (1.0 / NN) - mu * mu
        a = gam_ref[...] * lax.rsqrt(var + 1e-5)
        b = bet_ref[...] - mu * a
        a_ref[...] = a
        b_ref[...] = b
        c = cnt[...]
        c1 = jnp.maximum(c, 1.0)
        ep = ssum[...] / c1
        emb_ref[...] = jnp.where(c > 0.0, ep * a + b, 0.0)
        sv = jnp.maximum(ssq[...] / c1 - ep * ep, 0.0)
        std_ref[...] = jnp.abs(a) * jnp.sqrt(sv)


def _first_body(x_ref, seg_ref, w_ref, bias_ref, gam_ref, bet_ref,
                h_ref, emb_ref, std_ref, a_ref, b_ref,
                gsum, gsq, ssum, ssq, cnt):
    i = pl.program_id(0)
    z = x_ref[...]
    h = lax.dot_general(z, w_ref[...], (((1,), (1,)), ((), ())),
                        preferred_element_type=jnp.float32) + bias_ref[...]
    h_ref[...] = h
    _stats(i, seg_ref, h, emb_ref, std_ref, a_ref, b_ref, gam_ref, bet_ref,
           gsum, gsq, ssum, ssq, cnt)


def _layer_body(h_in_ref, a0_ref, a1_ref, seg_ref, w1_ref, w2_ref,
                gam_ref, bet_ref,
                h_ref, emb_ref, std_ref, a_ref, b_ref,
                gsum, gsq, ssum, ssq, cnt):
    i = pl.program_id(0)
    z = h_in_ref[...] + a0_ref[...] + a1_ref[...]
    dn = (((1,), (1,)), ((), ()))
    t = jnp.maximum(lax.dot_general(z, w1_ref[...], dn,
                                    preferred_element_type=jnp.float32), 0.0)
    h = jnp.maximum(lax.dot_general(t, w2_ref[...], dn,
                                    preferred_element_type=jnp.float32), 0.0)
    h_ref[...] = h
    _stats(i, seg_ref, h, emb_ref, std_ref, a_ref, b_ref, gam_ref, bet_ref,
           gsum, gsq, ssum, ssq, cnt)


_ROW = lambda i: (i, 0)
_FIX = lambda i: (0, 0)
_STAGE_OUT_SHAPE = [
    jax.ShapeDtypeStruct((NP, F), jnp.float32),
    jax.ShapeDtypeStruct((G, F), jnp.float32),
    jax.ShapeDtypeStruct((G, F), jnp.float32),
    jax.ShapeDtypeStruct((1, F), jnp.float32),
    jax.ShapeDtypeStruct((1, F), jnp.float32),
]
_STAGE_OUT_SPECS = [
    pl.BlockSpec((TILE, F), _ROW),
    pl.BlockSpec((G, F), _FIX),
    pl.BlockSpec((G, F), _FIX),
    pl.BlockSpec((1, F), _FIX),
    pl.BlockSpec((1, F), _FIX),
]
_STAGE_SCRATCH = [
    pltpu.VMEM((1, F), jnp.float32),
    pltpu.VMEM((1, F), jnp.float32),
    pltpu.VMEM((G, F), jnp.float32),
    pltpu.VMEM((G, F), jnp.float32),
    pltpu.VMEM((G, 1), jnp.float32),
]


def _make_first(interpret=False):
    return pl.pallas_call(
        _first_body,
        grid=(GRID,),
        in_specs=[
            pl.BlockSpec((TILE, F), _ROW),
            pl.BlockSpec((TILE, 1), _ROW),
            pl.BlockSpec((F, F), _FIX),
            pl.BlockSpec((1, F), _FIX),
            pl.BlockSpec((1, F), _FIX),
            pl.BlockSpec((1, F), _FIX),
        ],
        out_specs=_STAGE_OUT_SPECS,
        out_shape=_STAGE_OUT_SHAPE,
        scratch_shapes=_STAGE_SCRATCH,
        interpret=interpret,
    )


def _make_layer(interpret=False):
    return pl.pallas_call(
        _layer_body,
        grid=(GRID,),
        in_specs=[
            pl.BlockSpec((TILE, F), _ROW),
            pl.BlockSpec((TILE, F), _ROW),
            pl.BlockSpec((TILE, F), _ROW),
            pl.BlockSpec((TILE, 1), _ROW),
            pl.BlockSpec((F, F), _FIX),
            pl.BlockSpec((F, F), _FIX),
            pl.BlockSpec((1, F), _FIX),
            pl.BlockSpec((1, F), _FIX),
        ],
        out_specs=_STAGE_OUT_SPECS,
        out_shape=_STAGE_OUT_SHAPE,
        scratch_shapes=_STAGE_SCRATCH,
        interpret=interpret,
    )


def _norm_body(h_ref, a_ref, b_ref, o_ref):
    o_ref[...] = h_ref[...] * a_ref[...] + b_ref[...]


def _make_norm(interpret=False):
    return pl.pallas_call(
        _norm_body,
        grid=(GRID,),
        in_specs=[
            pl.BlockSpec((TILE, F), _ROW),
            pl.BlockSpec((1, F), _FIX),
            pl.BlockSpec((1, F), _FIX),
        ],
        out_specs=pl.BlockSpec((TILE, F), _ROW),
        out_shape=jax.ShapeDtypeStruct((NP, F), jnp.float32),
        interpret=interpret,
    )


_get_agg = functools.lru_cache(maxsize=None)(_make_agg)
_first_call = _make_first()
_layer_call = _make_layer()
_norm_call = _make_norm()


def kernel(x, edge_index, batch, Wt, bt, g0, beta0, W1s, W2s, gs, bs):
    f32 = jnp.float32
    xp = jnp.zeros((NP, F), f32).at[:NN].set(x)
    segp = jnp.full((NP, 1), G, jnp.int32).at[:NN, 0].set(batch)
    pad = EP - EE
    pad_dst = NN + (jnp.arange(pad, dtype=jnp.int32) % (NP - NN))
    pad_src = jnp.arange(pad, dtype=jnp.int32) % NN
    srcp = jnp.concatenate(
        [edge_index[0], pad_src]).reshape(NWORK, NSUP, SB, CK)
    dstp = jnp.concatenate(
        [edge_index[1], pad_dst]).reshape(NWORK, NSUP, SB, CK)
    zrows = jnp.zeros((RPS, F), f32)

    h_pre, emb, std, a, b = _first_call(
        xp, segp, Wt, bt.reshape(1, F), g0.reshape(1, F), beta0.reshape(1, F))
    embeds, stds = [emb], [std]
    for i in range(3):
        h_new = _norm_call(h_pre, a, b)
        aggs = _get_agg()(h_new, srcp, dstp, zrows)
        h_pre, emb, std, a, b = _layer_call(
            h_new, aggs[0], aggs[1], segp, W1s[i], W2s[i],
            gs[i].reshape(1, F), bs[i].reshape(1, F))
        embeds.append(emb)
        stds.append(std)
    return jnp.stack(embeds), jnp.stack(stds)


# speedup vs baseline: 3.5881x; 1.1942x over previous
import functools

import jax
import jax.numpy as jnp
from jax import lax
from jax.experimental import pallas as pl
from jax.experimental.pallas import tpu as pltpu
from jax.experimental.pallas import tpu_sc as plsc

F = 128
G = 64
NN = 10000
TILE = 512
GRID = 20
NP = TILE * GRID
EE = 320000
NWORK = 32
CK = 128
NB = 2
SB = 20
NSUP = 4
CH = SB * NSUP
EP = NWORK * CH * CK
RPS = NP // 16


def _sc_agg_body(h_hbm, src_hbm, dst_hbm, z_hbm, out_hbm,
                 src_v, dst_v, rows0, rows1, acc_sh,
                 rsem0, rsem1, isrc0, isrc1, idst0, idst1):
    rows = (rows0, rows1)
    rsems = (rsem0, rsem1)
    isrcs = (isrc0, isrc1)
    idsts = (idst0, idst1)
    c = lax.axis_index("c")
    sid = lax.axis_index("s")
    wid = sid * 2 + c
    pltpu.sync_copy(z_hbm, acc_sh.at[pl.ds(sid * RPS, RPS)])
    pltpu.sync_copy(src_hbm.at[wid, 0], src_v.at[0])
    pltpu.sync_copy(dst_hbm.at[wid, 0], dst_v.at[0])
    pltpu.async_copy(h_hbm.at[src_v.at[0, 0]], rows[0], rsems[0])
    plsc.subcore_barrier()

    for s in range(NSUP):
        slot = s % 2
        nxt = 1 - slot
        if s + 1 < NSUP:
            pltpu.async_copy(src_hbm.at[wid, s + 1], src_v.at[nxt],
                             isrcs[nxt])
            pltpu.async_copy(dst_hbm.at[wid, s + 1], dst_v.at[nxt],
                             idsts[nxt])

        def body(blk, carry, slot=slot):
            for b in range(NB):
                u = blk * NB + b
                pltpu.make_async_copy(h_hbm.at[src_v.at[slot, u]], rows[b],
                                      rsems[b]).wait()
                pltpu.async_copy(h_hbm.at[src_v.at[slot, u + 1]],
                                 rows[1 - b], rsems[1 - b])
                pltpu.sync_copy(rows[b], acc_sh.at[dst_v.at[slot, u]],
                                add=True)
            return carry

        lax.fori_loop(0, (SB - NB) // NB, body, 0)

        pltpu.make_async_copy(h_hbm.at[src_v.at[slot, SB - 2]], rows[0],
                              rsems[0]).wait()
        pltpu.async_copy(h_hbm.at[src_v.at[slot, SB - 1]], rows[1],
                         rsems[1])
        pltpu.sync_copy(rows[0], acc_sh.at[dst_v.at[slot, SB - 2]],
                        add=True)
        pltpu.make_async_copy(h_hbm.at[src_v.at[slot, SB - 1]], rows[1],
                              rsems[1]).wait()
        if s + 1 < NSUP:
            pltpu.make_async_copy(src_hbm.at[wid, 0], src_v.at[nxt],
                                  isrcs[nxt]).wait()
            pltpu.make_async_copy(dst_hbm.at[wid, 0], dst_v.at[nxt],
                                  idsts[nxt]).wait()
            pltpu.async_copy(h_hbm.at[src_v.at[nxt, 0]], rows[0], rsems[0])
        pltpu.sync_copy(rows[1], acc_sh.at[dst_v.at[slot, SB - 1]],
                        add=True)

    plsc.subcore_barrier()
    pltpu.sync_copy(acc_sh.at[pl.ds(sid * RPS, RPS)],
                    out_hbm.at[c, pl.ds(sid * RPS, RPS)])


def _make_agg(interpret=False):
    return pl.kernel(
        _sc_agg_body,
        out_type=jax.ShapeDtypeStruct((2, NP, F), jnp.float32),
        mesh=plsc.VectorSubcoreMesh(core_axis_name="c", subcore_axis_name="s",
                                    num_cores=2, num_subcores=16),
        scratch_types=[
            pltpu.VMEM((2, SB, CK), jnp.int32),
            pltpu.VMEM((2, SB, CK), jnp.int32),
            pltpu.VMEM((CK, F), jnp.float32),
            pltpu.VMEM((CK, F), jnp.float32),
            pltpu.VMEM_SHARED((NP, F), jnp.float32),
            pltpu.SemaphoreType.DMA,
            pltpu.SemaphoreType.DMA,
            pltpu.SemaphoreType.DMA,
            pltpu.SemaphoreType.DMA,
            pltpu.SemaphoreType.DMA,
            pltpu.SemaphoreType.DMA,
        ],
        interpret=interpret,
    )


def _stats(i, seg_ref, h, emb_ref, std_ref, a_ref, b_ref, gam_ref, bet_ref,
           gsum, gsq, ssum, ssq, cnt):
    @pl.when(i == 0)
    def _():
        gsum[...] = jnp.zeros_like(gsum)
        gsq[...] = jnp.zeros_like(gsq)
        ssum[...] = jnp.zeros_like(ssum)
        ssq[...] = jnp.zeros_like(ssq)
        cnt[...] = jnp.zeros_like(cnt)

    seg = seg_ref[...]
    gid = lax.broadcasted_iota(jnp.int32, (TILE, G), 1)
    mask = (seg == gid).astype(jnp.float32)
    rows = lax.broadcasted_iota(jnp.int32, (TILE, 1), 0) + i * TILE
    vm = (rows < NN).astype(jnp.float32)
    hm = h * vm
    gsum[...] += jnp.sum(hm, axis=0, keepdims=True)
    gsq[...] += jnp.sum(hm * h, axis=0, keepdims=True)
    dn = (((0,), (0,)), ((), ()))
    ssum[...] += lax.dot_general(mask, h, dn, preferred_element_type=jnp.float32)
    ssq[...] += lax.dot_general(mask, h * h, dn, preferred_element_type=jnp.float32)
    cnt[...] += lax.dot_general(mask, jnp.ones((TILE, 1), jnp.float32), dn,
                                preferred_element_type=jnp.float32)

    @pl.when(i == GRID - 1)
    def _():
        mu = gsum[...] * (1.0 / NN)
        var = gsq[...] * (1.0 / NN) - mu * mu
        a = gam_ref[...] * lax.rsqrt(var + 1e-5)
        b = bet_ref[...] - mu * a
        a_ref[...] = a
        b_ref[...] = b
        c = cnt[...]
        c1 = jnp.maximum(c, 1.0)
        ep = ssum[...] / c1
        emb_ref[...] = jnp.where(c > 0.0, ep * a + b, 0.0)
        sv = jnp.maximum(ssq[...] / c1 - ep * ep, 0.0)
        std_ref[...] = jnp.abs(a) * jnp.sqrt(sv)


def _first_body(x_ref, seg_ref, w_ref, bias_ref, gam_ref, bet_ref,
                h_ref, emb_ref, std_ref, a_ref, b_ref,
                gsum, gsq, ssum, ssq, cnt):
    i = pl.program_id(0)
    z = x_ref[...]
    h = lax.dot_general(z, w_ref[...], (((1,), (1,)), ((), ())),
                        preferred_element_type=jnp.float32) + bias_ref[...]
    h_ref[...] = h
    _stats(i, seg_ref, h, emb_ref, std_ref, a_ref, b_ref, gam_ref, bet_ref,
           gsum, gsq, ssum, ssq, cnt)


def _layer_body(h_in_ref, a0_ref, a1_ref, seg_ref, w1_ref, w2_ref,
                gam_ref, bet_ref,
                h_ref, emb_ref, std_ref, a_ref, b_ref,
                gsum, gsq, ssum, ssq, cnt):
    i = pl.program_id(0)
    z = h_in_ref[...] + a0_ref[0] + a1_ref[0]
    dn = (((1,), (1,)), ((), ()))
    t = jnp.maximum(lax.dot_general(z, w1_ref[...], dn,
                                    preferred_element_type=jnp.float32), 0.0)
    h = jnp.maximum(lax.dot_general(t, w2_ref[...], dn,
                                    preferred_element_type=jnp.float32), 0.0)
    h_ref[...] = h
    _stats(i, seg_ref, h, emb_ref, std_ref, a_ref, b_ref, gam_ref, bet_ref,
           gsum, gsq, ssum, ssq, cnt)


_ROW = lambda i: (i, 0)
_FIX = lambda i: (0, 0)
_STAGE_OUT_SHAPE = [
    jax.ShapeDtypeStruct((NP, F), jnp.float32),
    jax.ShapeDtypeStruct((G, F), jnp.float32),
    jax.ShapeDtypeStruct((G, F), jnp.float32),
    jax.ShapeDtypeStruct((1, F), jnp.float32),
    jax.ShapeDtypeStruct((1, F), jnp.float32),
]
_STAGE_OUT_SPECS = [
    pl.BlockSpec((TILE, F), _ROW),
    pl.BlockSpec((G, F), _FIX),
    pl.BlockSpec((G, F), _FIX),
    pl.BlockSpec((1, F), _FIX),
    pl.BlockSpec((1, F), _FIX),
]
_STAGE_SCRATCH = [
    pltpu.VMEM((1, F), jnp.float32),
    pltpu.VMEM((1, F), jnp.float32),
    pltpu.VMEM((G, F), jnp.float32),
    pltpu.VMEM((G, F), jnp.float32),
    pltpu.VMEM((G, 1), jnp.float32),
]


def _make_first(interpret=False):
    return pl.pallas_call(
        _first_body,
        grid=(GRID,),
        in_specs=[
            pl.BlockSpec((TILE, F), _ROW),
            pl.BlockSpec((TILE, 1), _ROW),
            pl.BlockSpec((F, F), _FIX),
            pl.BlockSpec((1, F), _FIX),
            pl.BlockSpec((1, F), _FIX),
            pl.BlockSpec((1, F), _FIX),
        ],
        out_specs=_STAGE_OUT_SPECS,
        out_shape=_STAGE_OUT_SHAPE,
        scratch_shapes=_STAGE_SCRATCH,
        interpret=interpret,
    )


def _make_layer(interpret=False):
    return pl.pallas_call(
        _layer_body,
        grid=(GRID,),
        in_specs=[
            pl.BlockSpec((TILE, F), _ROW),
            pl.BlockSpec((1, TILE, F), lambda i: (0, i, 0)),
            pl.BlockSpec((1, TILE, F), lambda i: (1, i, 0)),
            pl.BlockSpec((TILE, 1), _ROW),
            pl.BlockSpec((F, F), _FIX),
            pl.BlockSpec((F, F), _FIX),
            pl.BlockSpec((1, F), _FIX),
            pl.BlockSpec((1, F), _FIX),
        ],
        out_specs=_STAGE_OUT_SPECS,
        out_shape=_STAGE_OUT_SHAPE,
        scratch_shapes=_STAGE_SCRATCH,
        interpret=interpret,
    )


def _norm_body(h_ref, a_ref, b_ref, o_ref):
    o_ref[...] = h_ref[...] * a_ref[...] + b_ref[...]


def _make_norm(interpret=False):
    return pl.pallas_call(
        _norm_body,
        grid=(GRID,),
        in_specs=[
            pl.BlockSpec((TILE, F), _ROW),
            pl.BlockSpec((1, F), _FIX),
            pl.BlockSpec((1, F), _FIX),
        ],
        out_specs=pl.BlockSpec((TILE, F), _ROW),
        out_shape=jax.ShapeDtypeStruct((NP, F), jnp.float32),
        interpret=interpret,
    )


_get_agg = functools.lru_cache(maxsize=None)(_make_agg)
_first_call = _make_first()
_layer_call = _make_layer()
_norm_call = _make_norm()


def kernel(x, edge_index, batch, Wt, bt, g0, beta0, W1s, W2s, gs, bs):
    f32 = jnp.float32
    xp = jnp.zeros((NP, F), f32).at[:NN].set(x)
    segp = jnp.full((NP, 1), G, jnp.int32).at[:NN, 0].set(batch)
    pad = EP - EE
    pad_dst = NN + (jnp.arange(pad, dtype=jnp.int32) % (NP - NN))
    pad_src = jnp.arange(pad, dtype=jnp.int32) % NN
    srcp = jnp.concatenate(
        [edge_index[0], pad_src]).reshape(NWORK, NSUP, SB, CK)
    dstp = jnp.concatenate(
        [edge_index[1], pad_dst]).reshape(NWORK, NSUP, SB, CK)
    zrows = jnp.zeros((RPS, F), f32)

    h_pre, emb, std, a, b = _first_call(
        xp, segp, Wt, bt.reshape(1, F), g0.reshape(1, F), beta0.reshape(1, F))
    embeds, stds = [emb], [std]
    for i in range(3):
        h_new = _norm_call(h_pre, a, b)
        aggs = _get_agg()(h_new, srcp, dstp, zrows)
        h_pre, emb, std, a, b = _layer_call(
            h_new, aggs, aggs, segp, W1s[i], W2s[i],
            gs[i].reshape(1, F), bs[i].reshape(1, F))
        embeds.append(emb)
        stds.append(std)
    return jnp.stack(embeds), jnp.stack(stds)


# speedup vs baseline: 3.8790x; 1.0811x over previous
import functools

import jax
import jax.numpy as jnp
from jax import lax
from jax.experimental import pallas as pl
from jax.experimental.pallas import tpu as pltpu
from jax.experimental.pallas import tpu_sc as plsc

F = 128
G = 64
NN = 10000
TILE = 1024
GRID = 10
NP = TILE * GRID
EE = 320000
NWORK = 32
CK = 128
NB = 2
SB = 20
NSUP = 4
CH = SB * NSUP
EP = NWORK * CH * CK
RPS = NP // 16


def _sc_agg_body(h_hbm, src_hbm, dst_hbm, z_hbm, out_hbm,
                 src_v, dst_v, rows0, rows1, acc_sh,
                 rsem0, rsem1, isrc0, isrc1, idst0, idst1):
    rows = (rows0, rows1)
    rsems = (rsem0, rsem1)
    isrcs = (isrc0, isrc1)
    idsts = (idst0, idst1)
    c = lax.axis_index("c")
    sid = lax.axis_index("s")
    wid = sid * 2 + c
    pltpu.sync_copy(z_hbm, acc_sh.at[pl.ds(sid * RPS, RPS)])
    pltpu.sync_copy(src_hbm.at[wid, 0], src_v.at[0])
    pltpu.sync_copy(dst_hbm.at[wid, 0], dst_v.at[0])
    pltpu.async_copy(h_hbm.at[src_v.at[0, 0]], rows[0], rsems[0])
    plsc.subcore_barrier()

    for s in range(NSUP):
        slot = s % 2
        nxt = 1 - slot
        if s + 1 < NSUP:
            pltpu.async_copy(src_hbm.at[wid, s + 1], src_v.at[nxt],
                             isrcs[nxt])
            pltpu.async_copy(dst_hbm.at[wid, s + 1], dst_v.at[nxt],
                             idsts[nxt])

        def body(blk, carry, slot=slot):
            for b in range(NB):
                u = blk * NB + b
                pltpu.make_async_copy(h_hbm.at[src_v.at[slot, u]], rows[b],
                                      rsems[b]).wait()
                pltpu.async_copy(h_hbm.at[src_v.at[slot, u + 1]],
                                 rows[1 - b], rsems[1 - b])
                pltpu.sync_copy(rows[b], acc_sh.at[dst_v.at[slot, u]],
                                add=True)
            return carry

        lax.fori_loop(0, (SB - NB) // NB, body, 0)

        pltpu.make_async_copy(h_hbm.at[src_v.at[slot, SB - 2]], rows[0],
                              rsems[0]).wait()
        pltpu.async_copy(h_hbm.at[src_v.at[slot, SB - 1]], rows[1],
                         rsems[1])
        pltpu.sync_copy(rows[0], acc_sh.at[dst_v.at[slot, SB - 2]],
                        add=True)
        pltpu.make_async_copy(h_hbm.at[src_v.at[slot, SB - 1]], rows[1],
                              rsems[1]).wait()
        if s + 1 < NSUP:
            pltpu.make_async_copy(src_hbm.at[wid, 0], src_v.at[nxt],
                                  isrcs[nxt]).wait()
            pltpu.make_async_copy(dst_hbm.at[wid, 0], dst_v.at[nxt],
                                  idsts[nxt]).wait()
            pltpu.async_copy(h_hbm.at[src_v.at[nxt, 0]], rows[0], rsems[0])
        pltpu.sync_copy(rows[1], acc_sh.at[dst_v.at[slot, SB - 1]],
                        add=True)

    plsc.subcore_barrier()
    pltpu.sync_copy(acc_sh.at[pl.ds(sid * RPS, RPS)],
                    out_hbm.at[c, pl.ds(sid * RPS, RPS)])


def _make_agg(interpret=False):
    return pl.kernel(
        _sc_agg_body,
        out_type=jax.ShapeDtypeStruct((2, NP, F), jnp.float32),
        mesh=plsc.VectorSubcoreMesh(core_axis_name="c", subcore_axis_name="s",
                                    num_cores=2, num_subcores=16),
        scratch_types=[
            pltpu.VMEM((2, SB, CK), jnp.int32),
            pltpu.VMEM((2, SB, CK), jnp.int32),
            pltpu.VMEM((CK, F), jnp.float32),
            pltpu.VMEM((CK, F), jnp.float32),
            pltpu.VMEM_SHARED((NP, F), jnp.float32),
            pltpu.SemaphoreType.DMA,
            pltpu.SemaphoreType.DMA,
            pltpu.SemaphoreType.DMA,
            pltpu.SemaphoreType.DMA,
            pltpu.SemaphoreType.DMA,
            pltpu.SemaphoreType.DMA,
        ],
        interpret=interpret,
    )


def _stats(i, seg_ref, h, emb_ref, std_ref, a_ref, b_ref, gam_ref, bet_ref,
           gsum, gsq, ssum, ssq, cnt):
    @pl.when(i == 0)
    def _():
        gsum[...] = jnp.zeros_like(gsum)
        gsq[...] = jnp.zeros_like(gsq)
        ssum[...] = jnp.zeros_like(ssum)
        ssq[...] = jnp.zeros_like(ssq)
        cnt[...] = jnp.zeros_like(cnt)

    seg = seg_ref[...]
    gid = lax.broadcasted_iota(jnp.int32, (TILE, G), 1)
    mask = (seg == gid).astype(jnp.float32)
    rows = lax.broadcasted_iota(jnp.int32, (TILE, 1), 0) + i * TILE
    vm = (rows < NN).astype(jnp.float32)
    hm = h * vm
    gsum[...] += jnp.sum(hm, axis=0, keepdims=True)
    gsq[...] += jnp.sum(hm * h, axis=0, keepdims=True)
    dn = (((0,), (0,)), ((), ()))
    ssum[...] += lax.dot_general(mask, h, dn, preferred_element_type=jnp.float32)
    ssq[...] += lax.dot_general(mask, h * h, dn, preferred_element_type=jnp.float32)
    cnt[...] += lax.dot_general(mask, jnp.ones((TILE, 1), jnp.float32), dn,
                                preferred_element_type=jnp.float32)

    @pl.when(i == GRID - 1)
    def _():
        mu = gsum[...] * (1.0 / NN)
        var = gsq[...] * (1.0 / NN) - mu * mu
        a = gam_ref[...] * lax.rsqrt(var + 1e-5)
        b = bet_ref[...] - mu * a
        a_ref[...] = a
        b_ref[...] = b
        c = cnt[...]
        c1 = jnp.maximum(c, 1.0)
        ep = ssum[...] / c1
        emb_ref[...] = jnp.where(c > 0.0, ep * a + b, 0.0)
        sv = jnp.maximum(ssq[...] / c1 - ep * ep, 0.0)
        std_ref[...] = jnp.abs(a) * jnp.sqrt(sv)


def _first_body(x_ref, seg_ref, w_ref, bias_ref, gam_ref, bet_ref,
                h_ref, emb_ref, std_ref, a_ref, b_ref,
                gsum, gsq, ssum, ssq, cnt):
    i = pl.program_id(0)
    z = x_ref[...]
    h = lax.dot_general(z, w_ref[...], (((1,), (1,)), ((), ())),
                        preferred_element_type=jnp.float32) + bias_ref[...]
    h_ref[...] = h
    _stats(i, seg_ref, h, emb_ref, std_ref, a_ref, b_ref, gam_ref, bet_ref,
           gsum, gsq, ssum, ssq, cnt)


def _layer_body(h_in_ref, a0_ref, a1_ref, seg_ref, w1_ref, w2_ref,
                gam_ref, bet_ref,
                h_ref, emb_ref, std_ref, a_ref, b_ref,
                gsum, gsq, ssum, ssq, cnt):
    i = pl.program_id(0)
    z = h_in_ref[...] + a0_ref[0] + a1_ref[0]
    dn = (((1,), (1,)), ((), ()))
    t = jnp.maximum(lax.dot_general(z, w1_ref[...], dn,
                                    preferred_element_type=jnp.float32), 0.0)
    h = jnp.maximum(lax.dot_general(t, w2_ref[...], dn,
                                    preferred_element_type=jnp.float32), 0.0)
    h_ref[...] = h
    _stats(i, seg_ref, h, emb_ref, std_ref, a_ref, b_ref, gam_ref, bet_ref,
           gsum, gsq, ssum, ssq, cnt)


_ROW = lambda i: (i, 0)
_FIX = lambda i: (0, 0)
_STAGE_OUT_SHAPE = [
    jax.ShapeDtypeStruct((NP, F), jnp.float32),
    jax.ShapeDtypeStruct((G, F), jnp.float32),
    jax.ShapeDtypeStruct((G, F), jnp.float32),
    jax.ShapeDtypeStruct((1, F), jnp.float32),
    jax.ShapeDtypeStruct((1, F), jnp.float32),
]
_STAGE_OUT_SPECS = [
    pl.BlockSpec((TILE, F), _ROW),
    pl.BlockSpec((G, F), _FIX),
    pl.BlockSpec((G, F), _FIX),
    pl.BlockSpec((1, F), _FIX),
    pl.BlockSpec((1, F), _FIX),
]
_STAGE_SCRATCH = [
    pltpu.VMEM((1, F), jnp.float32),
    pltpu.VMEM((1, F), jnp.float32),
    pltpu.VMEM((G, F), jnp.float32),
    pltpu.VMEM((G, F), jnp.float32),
    pltpu.VMEM((G, 1), jnp.float32),
]


def _make_first(interpret=False):
    return pl.pallas_call(
        _first_body,
        grid=(GRID,),
        in_specs=[
            pl.BlockSpec((TILE, F), _ROW),
            pl.BlockSpec((TILE, 1), _ROW),
            pl.BlockSpec((F, F), _FIX),
            pl.BlockSpec((1, F), _FIX),
            pl.BlockSpec((1, F), _FIX),
            pl.BlockSpec((1, F), _FIX),
        ],
        out_specs=_STAGE_OUT_SPECS,
        out_shape=_STAGE_OUT_SHAPE,
        scratch_shapes=_STAGE_SCRATCH,
        interpret=interpret,
    )


def _make_layer(interpret=False):
    return pl.pallas_call(
        _layer_body,
        grid=(GRID,),
        in_specs=[
            pl.BlockSpec((TILE, F), _ROW),
            pl.BlockSpec((1, TILE, F), lambda i: (0, i, 0)),
            pl.BlockSpec((1, TILE, F), lambda i: (1, i, 0)),
            pl.BlockSpec((TILE, 1), _ROW),
            pl.BlockSpec((F, F), _FIX),
            pl.BlockSpec((F, F), _FIX),
            pl.BlockSpec((1, F), _FIX),
            pl.BlockSpec((1, F), _FIX),
        ],
        out_specs=_STAGE_OUT_SPECS,
        out_shape=_STAGE_OUT_SHAPE,
        scratch_shapes=_STAGE_SCRATCH,
        interpret=interpret,
    )


def _norm_body(h_ref, a_ref, b_ref, o_ref):
    o_ref[...] = h_ref[...] * a_ref[...] + b_ref[...]


def _make_norm(interpret=False):
    return pl.pallas_call(
        _norm_body,
        grid=(GRID,),
        in_specs=[
            pl.BlockSpec((TILE, F), _ROW),
            pl.BlockSpec((1, F), _FIX),
            pl.BlockSpec((1, F), _FIX),
        ],
        out_specs=pl.BlockSpec((TILE, F), _ROW),
        out_shape=jax.ShapeDtypeStruct((NP, F), jnp.float32),
        interpret=interpret,
    )


_get_agg = functools.lru_cache(maxsize=None)(_make_agg)
_first_call = _make_first()
_layer_call = _make_layer()
_norm_call = _make_norm()


def kernel(x, edge_index, batch, Wt, bt, g0, beta0, W1s, W2s, gs, bs):
    f32 = jnp.float32
    xp = jnp.zeros((NP, F), f32).at[:NN].set(x)
    segp = jnp.full((NP, 1), G, jnp.int32).at[:NN, 0].set(batch)
    pad = EP - EE
    pad_dst = NN + (jnp.arange(pad, dtype=jnp.int32) % (NP - NN))
    pad_src = jnp.arange(pad, dtype=jnp.int32) % NN
    srcp = jnp.concatenate(
        [edge_index[0], pad_src]).reshape(NWORK, NSUP, SB, CK)
    dstp = jnp.concatenate(
        [edge_index[1], pad_dst]).reshape(NWORK, NSUP, SB, CK)
    zrows = jnp.zeros((RPS, F), f32)

    h_pre, emb, std, a, b = _first_call(
        xp, segp, Wt, bt.reshape(1, F), g0.reshape(1, F), beta0.reshape(1, F))
    embeds, stds = [emb], [std]
    for i in range(3):
        h_new = _norm_call(h_pre, a, b)
        aggs = _get_agg()(h_new, srcp, dstp, zrows)
        h_pre, emb, std, a, b = _layer_call(
            h_new, aggs, aggs, segp, W1s[i], W2s[i],
            gs[i].reshape(1, F), bs[i].reshape(1, F))
        embeds.append(emb)
        stds.append(std)
    return jnp.stack(embeds), jnp.stack(stds)


# speedup vs baseline: 3.9759x; 1.0250x over previous
import functools

import jax
import jax.numpy as jnp
from jax import lax
from jax.experimental import pallas as pl
from jax.experimental.pallas import tpu as pltpu
from jax.experimental.pallas import tpu_sc as plsc

F = 128
G = 64
NN = 10000
TILE = 2048
GRID = 5
NP = TILE * GRID
EE = 320000
NWORK = 32
CK = 128
NB = 2
SB = 20
NSUP = 4
CH = SB * NSUP
EP = NWORK * CH * CK
RPS = NP // 16


def _sc_agg_body(h_hbm, src_hbm, dst_hbm, z_hbm, out_hbm,
                 src_v, dst_v, rows0, rows1, acc_sh,
                 rsem0, rsem1, isrc0, isrc1, idst0, idst1):
    rows = (rows0, rows1)
    rsems = (rsem0, rsem1)
    isrcs = (isrc0, isrc1)
    idsts = (idst0, idst1)
    c = lax.axis_index("c")
    sid = lax.axis_index("s")
    wid = sid * 2 + c
    pltpu.sync_copy(z_hbm, acc_sh.at[pl.ds(sid * RPS, RPS)])
    pltpu.sync_copy(src_hbm.at[wid, 0], src_v.at[0])
    pltpu.sync_copy(dst_hbm.at[wid, 0], dst_v.at[0])
    pltpu.async_copy(h_hbm.at[src_v.at[0, 0]], rows[0], rsems[0])
    plsc.subcore_barrier()

    for s in range(NSUP):
        slot = s % 2
        nxt = 1 - slot
        if s + 1 < NSUP:
            pltpu.async_copy(src_hbm.at[wid, s + 1], src_v.at[nxt],
                             isrcs[nxt])
            pltpu.async_copy(dst_hbm.at[wid, s + 1], dst_v.at[nxt],
                             idsts[nxt])

        def body(blk, carry, slot=slot):
            for b in range(NB):
                u = blk * NB + b
                pltpu.make_async_copy(h_hbm.at[src_v.at[slot, u]], rows[b],
                                      rsems[b]).wait()
                pltpu.async_copy(h_hbm.at[src_v.at[slot, u + 1]],
                                 rows[1 - b], rsems[1 - b])
                pltpu.sync_copy(rows[b], acc_sh.at[dst_v.at[slot, u]],
                                add=True)
            return carry

        lax.fori_loop(0, (SB - NB) // NB, body, 0)

        pltpu.make_async_copy(h_hbm.at[src_v.at[slot, SB - 2]], rows[0],
                              rsems[0]).wait()
        pltpu.async_copy(h_hbm.at[src_v.at[slot, SB - 1]], rows[1],
                         rsems[1])
        pltpu.sync_copy(rows[0], acc_sh.at[dst_v.at[slot, SB - 2]],
                        add=True)
        pltpu.make_async_copy(h_hbm.at[src_v.at[slot, SB - 1]], rows[1],
                              rsems[1]).wait()
        if s + 1 < NSUP:
            pltpu.make_async_copy(src_hbm.at[wid, 0], src_v.at[nxt],
                                  isrcs[nxt]).wait()
            pltpu.make_async_copy(dst_hbm.at[wid, 0], dst_v.at[nxt],
                                  idsts[nxt]).wait()
            pltpu.async_copy(h_hbm.at[src_v.at[nxt, 0]], rows[0], rsems[0])
        pltpu.sync_copy(rows[1], acc_sh.at[dst_v.at[slot, SB - 1]],
                        add=True)

    plsc.subcore_barrier()
    pltpu.sync_copy(acc_sh.at[pl.ds(sid * RPS, RPS)],
                    out_hbm.at[c, pl.ds(sid * RPS, RPS)])


def _make_agg(interpret=False):
    return pl.kernel(
        _sc_agg_body,
        out_type=jax.ShapeDtypeStruct((2, NP, F), jnp.float32),
        mesh=plsc.VectorSubcoreMesh(core_axis_name="c", subcore_axis_name="s",
                                    num_cores=2, num_subcores=16),
        scratch_types=[
            pltpu.VMEM((2, SB, CK), jnp.int32),
            pltpu.VMEM((2, SB, CK), jnp.int32),
            pltpu.VMEM((CK, F), jnp.float32),
            pltpu.VMEM((CK, F), jnp.float32),
            pltpu.VMEM_SHARED((NP, F), jnp.float32),
            pltpu.SemaphoreType.DMA,
            pltpu.SemaphoreType.DMA,
            pltpu.SemaphoreType.DMA,
            pltpu.SemaphoreType.DMA,
            pltpu.SemaphoreType.DMA,
            pltpu.SemaphoreType.DMA,
        ],
        interpret=interpret,
    )


def _stats(i, seg_ref, h, emb_ref, std_ref, a_ref, b_ref, gam_ref, bet_ref,
           gsum, gsq, ssum, ssq, cnt):
    @pl.when(i == 0)
    def _():
        gsum[...] = jnp.zeros_like(gsum)
        gsq[...] = jnp.zeros_like(gsq)
        ssum[...] = jnp.zeros_like(ssum)
        ssq[...] = jnp.zeros_like(ssq)
        cnt[...] = jnp.zeros_like(cnt)

    seg = seg_ref[...]
    gid = lax.broadcasted_iota(jnp.int32, (TILE, G), 1)
    mask = (seg == gid).astype(jnp.float32)
    rows = lax.broadcasted_iota(jnp.int32, (TILE, 1), 0) + i * TILE
    vm = (rows < NN).astype(jnp.float32)
    hm = h * vm
    gsum[...] += jnp.sum(hm, axis=0, keepdims=True)
    gsq[...] += jnp.sum(hm * h, axis=0, keepdims=True)
    dn = (((0,), (0,)), ((), ()))
    ssum[...] += lax.dot_general(mask, h, dn, preferred_element_type=jnp.float32)
    ssq[...] += lax.dot_general(mask, h * h, dn, preferred_element_type=jnp.float32)
    cnt[...] += lax.dot_general(mask, jnp.ones((TILE, 1), jnp.float32), dn,
                                preferred_element_type=jnp.float32)

    @pl.when(i == GRID - 1)
    def _():
        mu = gsum[...] * (1.0 / NN)
        var = gsq[...] * (1.0 / NN) - mu * mu
        a = gam_ref[...] * lax.rsqrt(var + 1e-5)
        b = bet_ref[...] - mu * a
        a_ref[...] = a
        b_ref[...] = b
        c = cnt[...]
        c1 = jnp.maximum(c, 1.0)
        ep = ssum[...] / c1
        emb_ref[...] = jnp.where(c > 0.0, ep * a + b, 0.0)
        sv = jnp.maximum(ssq[...] / c1 - ep * ep, 0.0)
        std_ref[...] = jnp.abs(a) * jnp.sqrt(sv)


def _first_body(x_ref, seg_ref, w_ref, bias_ref, gam_ref, bet_ref,
                h_ref, emb_ref, std_ref, a_ref, b_ref,
                gsum, gsq, ssum, ssq, cnt):
    i = pl.program_id(0)
    z = x_ref[...]
    h = lax.dot_general(z, w_ref[...], (((1,), (1,)), ((), ())),
                        preferred_element_type=jnp.float32) + bias_ref[...]
    h_ref[...] = h
    _stats(i, seg_ref, h, emb_ref, std_ref, a_ref, b_ref, gam_ref, bet_ref,
           gsum, gsq, ssum, ssq, cnt)


def _layer_body(h_in_ref, a0_ref, a1_ref, seg_ref, w1_ref, w2_ref,
                gam_ref, bet_ref,
                h_ref, emb_ref, std_ref, a_ref, b_ref,
                gsum, gsq, ssum, ssq, cnt):
    i = pl.program_id(0)
    z = h_in_ref[...] + a0_ref[0] + a1_ref[0]
    dn = (((1,), (1,)), ((), ()))
    t = jnp.maximum(lax.dot_general(z, w1_ref[...], dn,
                                    preferred_element_type=jnp.float32), 0.0)
    h = jnp.maximum(lax.dot_general(t, w2_ref[...], dn,
                                    preferred_element_type=jnp.float32), 0.0)
    h_ref[...] = h
    _stats(i, seg_ref, h, emb_ref, std_ref, a_ref, b_ref, gam_ref, bet_ref,
           gsum, gsq, ssum, ssq, cnt)


_ROW = lambda i: (i, 0)
_FIX = lambda i: (0, 0)
_STAGE_OUT_SHAPE = [
    jax.ShapeDtypeStruct((NP, F), jnp.float32),
    jax.ShapeDtypeStruct((G, F), jnp.float32),
    jax.ShapeDtypeStruct((G, F), jnp.float32),
    jax.ShapeDtypeStruct((1, F), jnp.float32),
    jax.ShapeDtypeStruct((1, F), jnp.float32),
]
_STAGE_OUT_SPECS = [
    pl.BlockSpec((TILE, F), _ROW),
    pl.BlockSpec((G, F), _FIX),
    pl.BlockSpec((G, F), _FIX),
    pl.BlockSpec((1, F), _FIX),
    pl.BlockSpec((1, F), _FIX),
]
_STAGE_SCRATCH = [
    pltpu.VMEM((1, F), jnp.float32),
    pltpu.VMEM((1, F), jnp.float32),
    pltpu.VMEM((G, F), jnp.float32),
    pltpu.VMEM((G, F), jnp.float32),
    pltpu.VMEM((G, 1), jnp.float32),
]


def _make_first(interpret=False):
    return pl.pallas_call(
        _first_body,
        grid=(GRID,),
        in_specs=[
            pl.BlockSpec((TILE, F), _ROW),
            pl.BlockSpec((TILE, 1), _ROW),
            pl.BlockSpec((F, F), _FIX),
            pl.BlockSpec((1, F), _FIX),
            pl.BlockSpec((1, F), _FIX),
            pl.BlockSpec((1, F), _FIX),
        ],
        out_specs=_STAGE_OUT_SPECS,
        out_shape=_STAGE_OUT_SHAPE,
        scratch_shapes=_STAGE_SCRATCH,
        interpret=interpret,
    )


def _make_layer(interpret=False):
    return pl.pallas_call(
        _layer_body,
        grid=(GRID,),
        in_specs=[
            pl.BlockSpec((TILE, F), _ROW),
            pl.BlockSpec((1, TILE, F), lambda i: (0, i, 0)),
            pl.BlockSpec((1, TILE, F), lambda i: (1, i, 0)),
            pl.BlockSpec((TILE, 1), _ROW),
            pl.BlockSpec((F, F), _FIX),
            pl.BlockSpec((F, F), _FIX),
            pl.BlockSpec((1, F), _FIX),
            pl.BlockSpec((1, F), _FIX),
        ],
        out_specs=_STAGE_OUT_SPECS,
        out_shape=_STAGE_OUT_SHAPE,
        scratch_shapes=_STAGE_SCRATCH,
        interpret=interpret,
    )


def _norm_body(h_ref, a_ref, b_ref, o_ref):
    o_ref[...] = h_ref[...] * a_ref[...] + b_ref[...]


def _make_norm(interpret=False):
    return pl.pallas_call(
        _norm_body,
        grid=(GRID,),
        in_specs=[
            pl.BlockSpec((TILE, F), _ROW),
            pl.BlockSpec((1, F), _FIX),
            pl.BlockSpec((1, F), _FIX),
        ],
        out_specs=pl.BlockSpec((TILE, F), _ROW),
        out_shape=jax.ShapeDtypeStruct((NP, F), jnp.float32),
        interpret=interpret,
    )


_get_agg = functools.lru_cache(maxsize=None)(_make_agg)
_first_call = _make_first()
_layer_call = _make_layer()
_norm_call = _make_norm()


def kernel(x, edge_index, batch, Wt, bt, g0, beta0, W1s, W2s, gs, bs):
    f32 = jnp.float32
    xp = jnp.zeros((NP, F), f32).at[:NN].set(x)
    segp = jnp.full((NP, 1), G, jnp.int32).at[:NN, 0].set(batch)
    pad = EP - EE
    pad_dst = NN + (jnp.arange(pad, dtype=jnp.int32) % (NP - NN))
    pad_src = jnp.arange(pad, dtype=jnp.int32) % NN
    srcp = jnp.concatenate(
        [edge_index[0], pad_src]).reshape(NWORK, NSUP, SB, CK)
    dstp = jnp.concatenate(
        [edge_index[1], pad_dst]).reshape(NWORK, NSUP, SB, CK)
    zrows = jnp.zeros((RPS, F), f32)

    h_pre, emb, std, a, b = _first_call(
        xp, segp, Wt, bt.reshape(1, F), g0.reshape(1, F), beta0.reshape(1, F))
    embeds, stds = [emb], [std]
    for i in range(3):
        h_new = _norm_call(h_pre, a, b)
        aggs = _get_agg()(h_new, srcp, dstp, zrows)
        h_pre, emb, std, a, b = _layer_call(
            h_new, aggs, aggs, segp, W1s[i], W2s[i],
            gs[i].reshape(1, F), bs[i].reshape(1, F))
        embeds.append(emb)
        stds.append(std)
    return jnp.stack(embeds), jnp.stack(stds)
